# Initial kernel scaffold; baseline (speedup 1.0000x reference)
import jax
import jax.numpy as jnp
from jax.experimental import pallas as pl


def kernel(inputs, weight):
    raise NotImplementedError("write your pallas kernel here")


# speedup vs baseline: 6.3329x; 6.3329x over previous
import functools

import jax
import jax.numpy as jnp
from jax import lax
from jax.experimental import pallas as pl
from jax.experimental.pallas import tpu as pltpu
from jax.experimental.pallas import tpu_sc as plsc


_D = 128
_CHUNK = 128


@functools.cache
def _build(num_idx: int, vocab: int, d: int):
    info = plsc.get_sparse_core_info()
    nw = info.num_cores * info.num_subcores
    assert num_idx % (nw * _CHUNK) == 0
    nchunk = num_idx // (nw * _CHUNK)
    mesh = plsc.VectorSubcoreMesh(core_axis_name="c", subcore_axis_name="s")

    @functools.partial(
        pl.kernel,
        out_type=jax.ShapeDtypeStruct((num_idx, d), jnp.float32),
        mesh=mesh,
        scratch_types=[
            pltpu.VMEM((nchunk, _CHUNK), jnp.int32),
            pltpu.VMEM((_CHUNK, d), jnp.float32),
            pltpu.SemaphoreType.DMA,
        ],
    )
    def emb(idx_hbm, table_hbm, out_hbm, idx_v, rows_v, sem):
        wid = lax.axis_index("s") * info.num_cores + lax.axis_index("c")
        base = wid * (nchunk * _CHUNK)
        pltpu.sync_copy(idx_hbm.at[wid], idx_v)

        def body(j, carry):
            pltpu.async_copy(table_hbm.at[idx_v.at[j]], rows_v, sem).wait()
            pltpu.sync_copy(rows_v, out_hbm.at[pl.ds(base + j * _CHUNK, _CHUNK)])
            return carry

        lax.fori_loop(0, nchunk, body, 0)

    return emb, nw, nchunk


def kernel(inputs, weight):
    b, t = inputs.shape
    vocab, d = weight.shape
    num_idx = b * t
    emb, nw, nchunk = _build(num_idx, vocab, d)
    idx = inputs.reshape(nw, nchunk, _CHUNK).astype(jnp.int32)
    out = emb(idx, weight)
    return out.reshape(b, t, d)


# speedup vs baseline: 9.2308x; 1.4576x over previous
import functools

import jax
import jax.numpy as jnp
from jax import lax
from jax.experimental import pallas as pl
from jax.experimental.pallas import tpu as pltpu
from jax.experimental.pallas import tpu_sc as plsc


_D = 128
_CHUNK = 128


@functools.cache
def _build(num_idx: int, vocab: int, d: int):
    info = plsc.get_sparse_core_info()
    nw = info.num_cores * info.num_subcores
    assert num_idx % (nw * _CHUNK) == 0
    nchunk = num_idx // (nw * _CHUNK)
    mesh = plsc.VectorSubcoreMesh(core_axis_name="c", subcore_axis_name="s")

    nbuf = 4
    look = 2
    assert nchunk % nbuf == 0 and nchunk > nbuf

    @functools.partial(
        pl.kernel,
        out_type=jax.ShapeDtypeStruct((num_idx, d), jnp.float32),
        mesh=mesh,
        scratch_types=[
            pltpu.VMEM((nchunk, _CHUNK), jnp.int32),
            pltpu.VMEM((nbuf, _CHUNK, d), jnp.float32),
            [pltpu.SemaphoreType.DMA] * nbuf,
            [pltpu.SemaphoreType.DMA] * nbuf,
        ],
    )
    def emb(idx_hbm, table_hbm, out_hbm, idx_v, rows_v, gsem, wsem):
        wid = lax.axis_index("s") * info.num_cores + lax.axis_index("c")
        base = wid * (nchunk * _CHUNK)
        pltpu.sync_copy(idx_hbm.at[wid], idx_v)

        def gather(j, b):
            pltpu.async_copy(table_hbm.at[idx_v.at[j]], rows_v.at[b], gsem[b])

        def writeback(j, b):
            pltpu.async_copy(
                rows_v.at[b], out_hbm.at[pl.ds(base + j * _CHUNK, _CHUNK)], wsem[b]
            )

        for j in range(look):
            gather(j, j)

        def outer(i, carry):
            j0 = i * nbuf
            for b in range(nbuf):
                j = j0 + b
                bg = (b + look) % nbuf
                @pl.when(j >= nbuf - look)
                def _():
                    pltpu.make_async_copy(
                        rows_v.at[bg], out_hbm.at[pl.ds(base, _CHUNK)], wsem[bg]
                    ).wait()

                @pl.when(j + look < nchunk)
                def _():
                    gather(j + look, bg)

                pltpu.make_async_copy(
                    table_hbm.at[idx_v.at[j]], rows_v.at[b], gsem[b]
                ).wait()
                writeback(j, b)
            return carry

        lax.fori_loop(0, nchunk // nbuf, outer, 0)
        for j in range(nchunk - nbuf + look, nchunk):
            b = j % nbuf
            pltpu.make_async_copy(
                rows_v.at[b], out_hbm.at[pl.ds(base, _CHUNK)], wsem[b]
            ).wait()

    return emb, nw, nchunk


def kernel(inputs, weight):
    b, t = inputs.shape
    vocab, d = weight.shape
    num_idx = b * t
    emb, nw, nchunk = _build(num_idx, vocab, d)
    idx = inputs.reshape(nw, nchunk, _CHUNK).astype(jnp.int32)
    out = emb(idx, weight)
    return out.reshape(b, t, d)


# speedup vs baseline: 9.2442x; 1.0015x over previous
import functools

import jax
import jax.numpy as jnp
from jax import lax
from jax.experimental import pallas as pl
from jax.experimental.pallas import tpu as pltpu
from jax.experimental.pallas import tpu_sc as plsc


_D = 128
_CHUNK = 128


@functools.cache
def _build(num_idx: int, vocab: int, d: int):
    info = plsc.get_sparse_core_info()
    nw = info.num_cores * info.num_subcores
    assert num_idx % (nw * _CHUNK) == 0
    nchunk = num_idx // (nw * _CHUNK)
    mesh = plsc.VectorSubcoreMesh(core_axis_name="c", subcore_axis_name="s")

    nbuf = 5
    look = 3
    assert nchunk % nbuf == 0 and nchunk > nbuf

    @functools.partial(
        pl.kernel,
        out_type=jax.ShapeDtypeStruct((num_idx, d), jnp.float32),
        mesh=mesh,
        scratch_types=[
            pltpu.VMEM((nchunk, _CHUNK), jnp.int32),
            pltpu.VMEM((nbuf, _CHUNK, d), jnp.float32),
            [pltpu.SemaphoreType.DMA] * nbuf,
            [pltpu.SemaphoreType.DMA] * nbuf,
        ],
    )
    def emb(idx_hbm, table_hbm, out_hbm, idx_v, rows_v, gsem, wsem):
        wid = lax.axis_index("s") * info.num_cores + lax.axis_index("c")
        base = wid * (nchunk * _CHUNK)
        pltpu.sync_copy(idx_hbm.at[wid], idx_v)

        def gather(j, b):
            pltpu.async_copy(table_hbm.at[idx_v.at[j]], rows_v.at[b], gsem[b])

        def writeback(j, b):
            pltpu.async_copy(
                rows_v.at[b], out_hbm.at[pl.ds(base + j * _CHUNK, _CHUNK)], wsem[b]
            )

        for j in range(look):
            gather(j, j)

        def outer(i, carry):
            j0 = i * nbuf
            for b in range(nbuf):
                j = j0 + b
                bg = (b + look) % nbuf
                @pl.when(j >= nbuf - look)
                def _():
                    pltpu.make_async_copy(
                        rows_v.at[bg], out_hbm.at[pl.ds(base, _CHUNK)], wsem[bg]
                    ).wait()

                @pl.when(j + look < nchunk)
                def _():
                    gather(j + look, bg)

                pltpu.make_async_copy(
                    table_hbm.at[idx_v.at[j]], rows_v.at[b], gsem[b]
                ).wait()
                writeback(j, b)
            return carry

        lax.fori_loop(0, nchunk // nbuf, outer, 0)
        for j in range(nchunk - nbuf + look, nchunk):
            b = j % nbuf
            pltpu.make_async_copy(
                rows_v.at[b], out_hbm.at[pl.ds(base, _CHUNK)], wsem[b]
            ).wait()

    return emb, nw, nchunk


def kernel(inputs, weight):
    b, t = inputs.shape
    vocab, d = weight.shape
    num_idx = b * t
    emb, nw, nchunk = _build(num_idx, vocab, d)
    idx = inputs.reshape(nw, nchunk, _CHUNK).astype(jnp.int32)
    out = emb(idx, weight)
    return out.reshape(b, t, d)


# speedup vs baseline: 9.5991x; 1.0384x over previous
import functools

import jax
import jax.numpy as jnp
from jax import lax
from jax.experimental import pallas as pl
from jax.experimental.pallas import tpu as pltpu
from jax.experimental.pallas import tpu_sc as plsc


_D = 128
_CHUNK = 128


@functools.cache
def _build(num_idx: int, vocab: int, d: int):
    info = plsc.get_sparse_core_info()
    nw = info.num_cores * info.num_subcores
    ns = info.num_subcores
    nchunk = num_idx // (nw * _CHUNK)
    nbuf = 4
    nbufs = 2
    look = 2
    assert nchunk % nbuf == 0 and nchunk > nbuf
    mesh = plsc.VectorSubcoreMesh(core_axis_name="c", subcore_axis_name="s")

    @functools.partial(
        pl.kernel,
        out_type=jax.ShapeDtypeStruct((num_idx, d), jnp.float32),
        mesh=mesh,
        scratch_types=[
            pltpu.VMEM((nchunk, _CHUNK), jnp.int32),
            pltpu.VMEM((nbuf, _CHUNK, d), jnp.float32),
            pltpu.VMEM_SHARED((ns, nbufs, _CHUNK, d), jnp.float32),
            [pltpu.SemaphoreType.DMA] * nbuf,
            [pltpu.SemaphoreType.DMA] * nbuf,
            [pltpu.SemaphoreType.DMA] * nbufs,
        ],
    )
    def emb(idx_hbm, table_hbm, out_hbm, idx_v, rows_v, rows_s, gsem, xsem, wsem):
        cid = lax.axis_index("c")
        sid = lax.axis_index("s")
        wid = sid * info.num_cores + cid
        base = wid * (nchunk * _CHUNK)
        pltpu.sync_copy(idx_hbm.at[wid], idx_v)

        def gather(j, b):
            pltpu.async_copy(table_hbm.at[idx_v.at[j]], rows_v.at[b], gsem[b])

        for j in range(look):
            gather(j, j)

        def outer(i, carry):
            j0 = i * nbuf
            for b in range(nbuf):
                j = j0 + b
                bg = (b + look) % nbuf
                bp = (b + nbuf - 1) % nbuf
                bs = b % nbufs
                bsp = (b + nbuf - 1) % nbufs

                @pl.when(j >= nbufs)
                def _():
                    pltpu.make_async_copy(
                        rows_s.at[sid, bs], out_hbm.at[pl.ds(base, _CHUNK)], wsem[bs]
                    ).wait()

                pltpu.make_async_copy(
                    table_hbm.at[idx_v.at[j]], rows_v.at[b], gsem[b]
                ).wait()
                pltpu.async_copy(rows_v.at[b], rows_s.at[sid, bs], xsem[b])

                @pl.when(j >= 1)
                def _():
                    pltpu.make_async_copy(
                        rows_v.at[bp], rows_s.at[sid, bsp], xsem[bp]
                    ).wait()
                    pltpu.async_copy(
                        rows_s.at[sid, bsp],
                        out_hbm.at[pl.ds(base + (j - 1) * _CHUNK, _CHUNK)],
                        wsem[bsp],
                    )

                @pl.when(j + look < nchunk)
                def _():
                    gather(j + look, bg)

            return carry

        lax.fori_loop(0, nchunk // nbuf, outer, 0)
        bl = (nchunk - 1) % nbuf
        bls = (nchunk - 1) % nbufs
        pltpu.make_async_copy(rows_v.at[bl], rows_s.at[sid, bls], xsem[bl]).wait()
        pltpu.async_copy(
            rows_s.at[sid, bls],
            out_hbm.at[pl.ds(base + (nchunk - 1) * _CHUNK, _CHUNK)],
            wsem[bls],
        )
        for b in range(nbufs):
            pltpu.make_async_copy(
                rows_s.at[sid, b], out_hbm.at[pl.ds(base, _CHUNK)], wsem[b]
            ).wait()

    return emb, nw, nchunk


def kernel(inputs, weight):
    b, t = inputs.shape
    vocab, d = weight.shape
    num_idx = b * t
    emb, nw, nchunk = _build(num_idx, vocab, d)
    idx = inputs.reshape(nw, nchunk, _CHUNK).astype(jnp.int32)
    out = emb(idx, weight)
    return out.reshape(b, t, d)


# speedup vs baseline: 9.6150x; 1.0017x over previous
import functools

import jax
import jax.numpy as jnp
from jax import lax
from jax.experimental import pallas as pl
from jax.experimental.pallas import tpu as pltpu
from jax.experimental.pallas import tpu_sc as plsc


_D = 128
_CHUNK = 128


@functools.cache
def _build(num_idx: int, vocab: int, d: int):
    info = plsc.get_sparse_core_info()
    nw = info.num_cores * info.num_subcores
    ns = info.num_subcores
    nchunk = num_idx // (nw * _CHUNK)
    nbuf = 4
    nbufs = 2
    look = 3
    assert nchunk % nbuf == 0 and nchunk > nbuf
    mesh = plsc.VectorSubcoreMesh(core_axis_name="c", subcore_axis_name="s")

    @functools.partial(
        pl.kernel,
        out_type=jax.ShapeDtypeStruct((num_idx, d), jnp.float32),
        mesh=mesh,
        scratch_types=[
            pltpu.VMEM((nchunk, _CHUNK), jnp.int32),
            pltpu.VMEM((nbuf, _CHUNK, d), jnp.float32),
            pltpu.VMEM_SHARED((ns, nbufs, _CHUNK, d), jnp.float32),
            [pltpu.SemaphoreType.DMA] * nbuf,
            [pltpu.SemaphoreType.DMA] * nbuf,
            [pltpu.SemaphoreType.DMA] * nbufs,
        ],
    )
    def emb(idx_hbm, table_hbm, out_hbm, idx_v, rows_v, rows_s, gsem, xsem, wsem):
        cid = lax.axis_index("c")
        sid = lax.axis_index("s")
        wid = sid * info.num_cores + cid
        base = wid * (nchunk * _CHUNK)
        pltpu.sync_copy(idx_hbm.at[wid], idx_v)

        def gather(j, b):
            pltpu.async_copy(table_hbm.at[idx_v.at[j]], rows_v.at[b], gsem[b])

        for j in range(look):
            gather(j, j)

        def outer(i, carry):
            j0 = i * nbuf
            for b in range(nbuf):
                j = j0 + b
                bg = (b + look) % nbuf
                bp = (b + nbuf - 1) % nbuf
                bs = b % nbufs
                bsp = (b + nbuf - 1) % nbufs

                @pl.when(j >= nbufs)
                def _():
                    pltpu.make_async_copy(
                        rows_s.at[sid, bs], out_hbm.at[pl.ds(base, _CHUNK)], wsem[bs]
                    ).wait()

                pltpu.make_async_copy(
                    table_hbm.at[idx_v.at[j]], rows_v.at[b], gsem[b]
                ).wait()
                pltpu.async_copy(rows_v.at[b], rows_s.at[sid, bs], xsem[b])

                @pl.when(j >= 1)
                def _():
                    pltpu.make_async_copy(
                        rows_v.at[bp], rows_s.at[sid, bsp], xsem[bp]
                    ).wait()
                    pltpu.async_copy(
                        rows_s.at[sid, bsp],
                        out_hbm.at[pl.ds(base + (j - 1) * _CHUNK, _CHUNK)],
                        wsem[bsp],
                    )

                @pl.when(j + look < nchunk)
                def _():
                    gather(j + look, bg)

            return carry

        lax.fori_loop(0, nchunk // nbuf, outer, 0)
        bl = (nchunk - 1) % nbuf
        bls = (nchunk - 1) % nbufs
        pltpu.make_async_copy(rows_v.at[bl], rows_s.at[sid, bls], xsem[bl]).wait()
        pltpu.async_copy(
            rows_s.at[sid, bls],
            out_hbm.at[pl.ds(base + (nchunk - 1) * _CHUNK, _CHUNK)],
            wsem[bls],
        )
        for b in range(nbufs):
            pltpu.make_async_copy(
                rows_s.at[sid, b], out_hbm.at[pl.ds(base, _CHUNK)], wsem[b]
            ).wait()

    return emb, nw, nchunk


def kernel(inputs, weight):
    b, t = inputs.shape
    vocab, d = weight.shape
    num_idx = b * t
    emb, nw, nchunk = _build(num_idx, vocab, d)
    idx = inputs.reshape(nw, nchunk, _CHUNK).astype(jnp.int32)
    out = emb(idx, weight)
    return out.reshape(b, t, d)
